# baseline (speedup 1.0000x reference)
import jax
from jax.experimental import pallas as pl
from jax.experimental.pallas import tpu as pltpu

_CHUNK_ROWS = (3072, 5120)


def kernel(x, pe_weight):
    batch = x.shape[0]
    maxlen, d = pe_weight.shape
    assert sum(_CHUNK_ROWS) == maxlen
    offs = []
    o = 0
    for c in _CHUNK_ROWS:
        offs.append(o)
        o += c
    nchunk = len(_CHUNK_ROWS)

    def _body(w_hbm, out_hbm, buf, in_sems, out_sems):
        for i, (o, c) in enumerate(zip(offs, _CHUNK_ROWS)):
            pltpu.make_async_copy(
                w_hbm.at[pl.ds(o, c)],
                buf.at[pl.ds(o, c)],
                in_sems.at[i],
            ).start()
        for i, (o, c) in enumerate(zip(offs, _CHUNK_ROWS)):
            pltpu.make_async_copy(
                w_hbm.at[pl.ds(o, c)],
                buf.at[pl.ds(o, c)],
                in_sems.at[i],
            ).wait()
            for b in range(batch):
                pltpu.make_async_copy(
                    buf.at[pl.ds(o, c)],
                    out_hbm.at[b, pl.ds(o, c)],
                    out_sems.at[i, b],
                ).start()
        for i, (o, c) in enumerate(zip(offs, _CHUNK_ROWS)):
            for b in range(batch):
                pltpu.make_async_copy(
                    buf.at[pl.ds(o, c)],
                    out_hbm.at[b, pl.ds(o, c)],
                    out_sems.at[i, b],
                ).wait()

    return pl.pallas_call(
        _body,
        in_specs=[pl.BlockSpec(memory_space=pltpu.MemorySpace.HBM)],
        out_specs=pl.BlockSpec(memory_space=pltpu.MemorySpace.HBM),
        out_shape=jax.ShapeDtypeStruct((batch, maxlen, d), pe_weight.dtype),
        scratch_shapes=[
            pltpu.VMEM((maxlen, d), pe_weight.dtype),
            pltpu.SemaphoreType.DMA((nchunk,)),
            pltpu.SemaphoreType.DMA((nchunk, batch)),
        ],
    )(pe_weight)
